# Initial kernel scaffold; baseline (speedup 1.0000x reference)
import jax
import jax.numpy as jnp
from jax.experimental import pallas as pl


def kernel(outputs, tokens, tokens_lens):
    raise NotImplementedError("write your pallas kernel here")


# speedup vs baseline: 1.7880x; 1.7880x over previous
import jax
import jax.numpy as jnp
from jax.experimental import pallas as pl
from jax.experimental.pallas import tpu as pltpu


def _partial_body(lens_ref, x_ref, tgt_ref, part_ref):
    i = pl.program_id(0)
    x = x_ref[...]
    Tb, Bb, Vb = x.shape
    m = jnp.max(x, axis=-1)
    idx = jax.lax.broadcasted_iota(jnp.int32, x.shape, 2)
    cand = jnp.where(x == m[..., None], idx, Vb)
    pred = jnp.min(cand, axis=-1)
    tgt = tgt_ref[...]
    tids = i * Tb + jax.lax.broadcasted_iota(jnp.int32, (Tb, Bb), 0)
    mask = tids < (lens_ref[...] - 1)
    corr = jnp.logical_and(pred == tgt, mask)
    c = jnp.sum(corr.astype(jnp.float32))
    v = jnp.sum(mask.astype(jnp.float32))
    lane = jax.lax.broadcasted_iota(jnp.int32, (1, 128), 1)
    row = jnp.where(lane == 0, c, jnp.where(lane == 1, v, 0.0))
    part_ref[...] = row.reshape(1, 1, 128)


def _finish_body(part_ref, out_ref):
    p = part_ref[...].reshape(part_ref.shape[0], 128)
    lane = jax.lax.broadcasted_iota(jnp.int32, p.shape, 1)
    c = jnp.sum(jnp.where(lane == 0, p, 0.0))
    v = jnp.sum(jnp.where(lane == 1, p, 0.0))
    out_ref[...] = jnp.full((1, 128), c / v, dtype=jnp.float32)


def kernel(outputs, tokens, tokens_lens):
    T, B, V = outputs.shape
    Tb = 32
    n = T // Tb
    targets = jnp.roll(tokens, -1, axis=0)
    lens2d = tokens_lens.reshape(1, B)

    parts = pl.pallas_call(
        _partial_body,
        grid=(n,),
        in_specs=[
            pl.BlockSpec((1, B), lambda i: (0, 0)),
            pl.BlockSpec((Tb, B, V), lambda i: (i, 0, 0)),
            pl.BlockSpec((Tb, B), lambda i: (i, 0)),
        ],
        out_specs=pl.BlockSpec((1, 1, 128), lambda i: (i, 0, 0)),
        out_shape=jax.ShapeDtypeStruct((n, 1, 128), jnp.float32),
        compiler_params=pltpu.CompilerParams(
            dimension_semantics=("parallel",),
        ),
    )(lens2d, outputs, targets)

    acc = pl.pallas_call(
        _finish_body,
        out_shape=jax.ShapeDtypeStruct((1, 128), jnp.float32),
    )(parts)
    return acc[0, 0]


# speedup vs baseline: 1.9579x; 1.0950x over previous
import jax
import jax.numpy as jnp
from jax.experimental import pallas as pl
from jax.experimental.pallas import tpu as pltpu


def _partial_body(lens_ref, x_ref, tgt_ref, part_ref):
    i = pl.program_id(0)
    x = x_ref[...]
    Tb, Bb, Vb = x.shape
    m = jnp.max(x, axis=-1)
    idx = jax.lax.broadcasted_iota(jnp.int32, x.shape, 2)
    cand = jnp.where(x == m[..., None], idx, Vb)
    pred = jnp.min(cand, axis=-1)
    tgt = tgt_ref[...]
    tids = i * Tb + jax.lax.broadcasted_iota(jnp.int32, (Tb, Bb), 0)
    mask = tids < (lens_ref[...] - 1)
    corr = jnp.logical_and(pred == tgt, mask)
    c = jnp.sum(corr.astype(jnp.float32))
    v = jnp.sum(mask.astype(jnp.float32))
    lane = jax.lax.broadcasted_iota(jnp.int32, (1, 128), 1)
    row = jnp.where(lane == 0, c, jnp.where(lane == 1, v, 0.0))
    part_ref[...] = row.reshape(1, 1, 128)


def _finish_body(part_ref, out_ref):
    p = part_ref[...].reshape(part_ref.shape[0], 128)
    lane = jax.lax.broadcasted_iota(jnp.int32, p.shape, 1)
    c = jnp.sum(jnp.where(lane == 0, p, 0.0))
    v = jnp.sum(jnp.where(lane == 1, p, 0.0))
    out_ref[...] = jnp.full((1, 128), c / v, dtype=jnp.float32)


def kernel(outputs, tokens, tokens_lens):
    T, B, V = outputs.shape
    Tb = 64
    n = T // Tb
    targets = jnp.roll(tokens, -1, axis=0)
    lens2d = tokens_lens.reshape(1, B)

    parts = pl.pallas_call(
        _partial_body,
        grid=(n,),
        in_specs=[
            pl.BlockSpec((1, B), lambda i: (0, 0)),
            pl.BlockSpec((Tb, B, V), lambda i: (i, 0, 0)),
            pl.BlockSpec((Tb, B), lambda i: (i, 0)),
        ],
        out_specs=pl.BlockSpec((1, 1, 128), lambda i: (i, 0, 0)),
        out_shape=jax.ShapeDtypeStruct((n, 1, 128), jnp.float32),
        compiler_params=pltpu.CompilerParams(
            dimension_semantics=("parallel",),
        ),
    )(lens2d, outputs, targets)

    acc = pl.pallas_call(
        _finish_body,
        out_shape=jax.ShapeDtypeStruct((1, 128), jnp.float32),
    )(parts)
    return acc[0, 0]
